# baseline (speedup 1.0000x reference)
import functools

import jax
import jax.numpy as jnp
from jax import lax
from jax.experimental import pallas as pl
from jax.experimental.pallas import tpu as pltpu
from jax.experimental.pallas import tpu_sc as plsc

N_ROWS = 320000
N_FEAT = 128
N_SEG = 10000
N_CORES = 2
N_SUBCORES = 16
ROWS_PER_TILE = N_ROWS // N_SUBCORES
CHUNK = 128
N_CHUNKS = ROWS_PER_TILE // CHUNK
REM = ROWS_PER_TILE - N_CHUNKS * CHUNK
N_SEG_PAD = 10240
SEG_PER_TILE = N_SEG_PAD // N_SUBCORES


def _sc_body(x_hbm, ids_hbm, zs_hbm, on_hbm, out_s, out_c,
             idx_a, idx_b, rows_a, rows_b, idx_r, rows_r,
             sem_ia, sem_ib, sem_ra, sem_rb, sem_sa, sem_sb, acc_sh):
    c = lax.axis_index("c")
    s = lax.axis_index("s")
    seg0 = s * SEG_PER_TILE
    row_base = s * ROWS_PER_TILE
    idx_v = (idx_a, idx_b)
    rows_v = (rows_a, rows_b)
    sem_i = (sem_ia, sem_ib)
    sem_r = (sem_ra, sem_rb)
    sem_s = (sem_sa, sem_sb)

    pltpu.sync_copy(zs_hbm, acc_sh.at[pl.ds(seg0, SEG_PER_TILE)])
    pltpu.sync_copy(on_hbm, rows_a)
    pltpu.sync_copy(on_hbm, rows_b)
    pltpu.sync_copy(on_hbm.at[pl.ds(0, REM)], rows_r)
    plsc.subcore_barrier()

    def issue(i, b):
        base = row_base + i * CHUNK
        pltpu.async_copy(ids_hbm.at[pl.ds(base, CHUNK)], idx_v[b], sem_i[b])

        @pl.when(c == 0)
        def _():
            pltpu.async_copy(x_hbm.at[pl.ds(base, CHUNK)], rows_v[b], sem_r[b])

    def wait_slot(b):
        pltpu.make_async_copy(ids_hbm.at[pl.ds(0, CHUNK)], idx_v[b],
                              sem_i[b]).wait()

        @pl.when(c == 0)
        def _():
            pltpu.make_async_copy(x_hbm.at[pl.ds(0, CHUNK)], rows_v[b],
                                  sem_r[b]).wait()

    def wait_scatter(b):
        pltpu.make_async_copy(rows_v[b], acc_sh.at[idx_v[b]], sem_s[b]).wait()

    issue(0, 0)

    def step(g, _):
        for b in range(2):
            i = g * 2 + b
            wait_slot(b)
            pltpu.async_copy(rows_v[b], acc_sh.at[idx_v[b]], sem_s[b], add=True)

            @pl.when(i >= 1)
            def _():
                wait_scatter(b ^ 1)

            @pl.when(i + 1 < N_CHUNKS)
            def _():
                issue(i + 1, b ^ 1)
        return 0

    lax.fori_loop(0, N_CHUNKS // 2, step, 0)
    wait_scatter((N_CHUNKS - 1) % 2)

    rem_base = row_base + N_CHUNKS * CHUNK
    pltpu.sync_copy(ids_hbm.at[pl.ds(rem_base, REM)], idx_r)

    @pl.when(c == 0)
    def _():
        pltpu.sync_copy(x_hbm.at[pl.ds(rem_base, REM)], rows_r)

    pltpu.sync_copy(rows_r, acc_sh.at[idx_r], add=True)

    plsc.subcore_barrier()

    @pl.when(c == 0)
    def _():
        pltpu.sync_copy(acc_sh.at[pl.ds(seg0, SEG_PER_TILE)],
                        out_s.at[pl.ds(seg0, SEG_PER_TILE)])

    @pl.when(c == 1)
    def _():
        pltpu.sync_copy(acc_sh.at[pl.ds(seg0, SEG_PER_TILE)],
                        out_c.at[pl.ds(seg0, SEG_PER_TILE)])


_sc_accumulate = functools.partial(
    pl.kernel,
    out_type=(
        jax.ShapeDtypeStruct((N_SEG_PAD, N_FEAT), jnp.float32),
        jax.ShapeDtypeStruct((N_SEG_PAD, N_FEAT), jnp.float32),
    ),
    mesh=plsc.VectorSubcoreMesh(core_axis_name="c", subcore_axis_name="s"),
    scratch_types=(
        pltpu.VMEM((CHUNK,), jnp.int32),
        pltpu.VMEM((CHUNK,), jnp.int32),
        pltpu.VMEM((CHUNK, N_FEAT), jnp.float32),
        pltpu.VMEM((CHUNK, N_FEAT), jnp.float32),
        pltpu.VMEM((REM,), jnp.int32),
        pltpu.VMEM((REM, N_FEAT), jnp.float32),
        pltpu.SemaphoreType.DMA,
        pltpu.SemaphoreType.DMA,
        pltpu.SemaphoreType.DMA,
        pltpu.SemaphoreType.DMA,
        pltpu.SemaphoreType.DMA,
        pltpu.SemaphoreType.DMA,
        pltpu.VMEM_SHARED((N_SEG_PAD, N_FEAT), jnp.float32),
    ),
)(_sc_body)


def _combine_body(ps_ref, pc_ref, o_ref):
    o_ref[...] = ps_ref[...] / jnp.maximum(pc_ref[:, 0:1], 1.0)


_combine = pl.pallas_call(
    _combine_body,
    grid=(10,),
    in_specs=[
        pl.BlockSpec((1000, N_FEAT), lambda j: (j, 0)),
        pl.BlockSpec((1000, N_FEAT), lambda j: (j, 0)),
    ],
    out_specs=pl.BlockSpec((1000, N_FEAT), lambda j: (j, 0)),
    out_shape=jax.ShapeDtypeStruct((N_SEG, N_FEAT), jnp.float32),
)


@jax.jit
def kernel(inputs, unq_inv):
    ids = unq_inv.astype(jnp.int32)
    zs = jnp.zeros((SEG_PER_TILE, N_FEAT), jnp.float32)
    on = jnp.ones((CHUNK, N_FEAT), jnp.float32)
    sums, cnts = _sc_accumulate(inputs, ids, zs, on)
    return _combine(sums, cnts)


# speedup vs baseline: 1.5679x; 1.5679x over previous
import functools

import jax
import jax.numpy as jnp
from jax import lax
from jax.experimental import pallas as pl
from jax.experimental.pallas import tpu as pltpu
from jax.experimental.pallas import tpu_sc as plsc

N_ROWS = 320000
N_FEAT = 128
N_SEG = 10000
N_CORES = 2
N_SUBCORES = 16
N_TILES = N_CORES * N_SUBCORES
ROWS_PER_TILE = N_ROWS // N_TILES
CHUNK = 128
N_CHUNKS = ROWS_PER_TILE // CHUNK
REM = ROWS_PER_TILE - N_CHUNKS * CHUNK
N_SEG_PAD = 10240
SEG_PER_TILE = N_SEG_PAD // N_SUBCORES
N_GROUPS = CHUNK // 16


def _sc_body(x_hbm, ids_hbm, zs_hbm, z1_hbm, out_s, out_c,
             idx_a, idx_b, rows_a, rows_b, idx_r, rows_r,
             cnt_v,
             sem_ia, sem_ib, sem_ra, sem_rb, sem_sa, sem_sb,
             acc_sh):
    c = lax.axis_index("c")
    s = lax.axis_index("s")
    w = c * N_SUBCORES + s
    seg0 = s * SEG_PER_TILE
    row_base = w * ROWS_PER_TILE
    idx_v = (idx_a, idx_b)
    rows_v = (rows_a, rows_b)
    sem_i = (sem_ia, sem_ib)
    sem_r = (sem_ra, sem_rb)
    sem_s = (sem_sa, sem_sb)

    iota = lax.iota(jnp.int32, 16)
    shift_idx = jnp.maximum(iota - 1, 0)
    lane0 = iota == 0

    pltpu.sync_copy(zs_hbm, acc_sh.at[pl.ds(seg0, SEG_PER_TILE)])
    pltpu.sync_copy(z1_hbm, cnt_v)
    pltpu.sync_copy(ids_hbm.at[pl.ds(row_base, 16)], idx_r)
    carry0 = plsc.load_gather(idx_r, [jnp.zeros((16,), jnp.int32)])
    plsc.subcore_barrier()

    def issue(i, b):
        base = row_base + i * CHUNK
        pltpu.async_copy(ids_hbm.at[pl.ds(base, CHUNK)], idx_v[b], sem_i[b])
        pltpu.async_copy(x_hbm.at[pl.ds(base, CHUNK)], rows_v[b], sem_r[b])

    def wait_slot(b):
        pltpu.make_async_copy(ids_hbm.at[pl.ds(0, CHUNK)], idx_v[b],
                              sem_i[b]).wait()
        pltpu.make_async_copy(x_hbm.at[pl.ds(0, CHUNK)], rows_v[b],
                              sem_r[b]).wait()

    def wait_scatter(b):
        pltpu.make_async_copy(rows_v[b], acc_sh.at[idx_v[b]], sem_s[b]).wait()

    def count_group(g, p, base_val):
        m = g != p
        val = (base_val + iota).astype(jnp.float32)
        plsc.addupdate_scatter(cnt_v, [g], -val, mask=m)
        plsc.addupdate_scatter(cnt_v, [p], val, mask=m)

    def count_chunk(ids_ref, local_base, n_groups, carry):
        for j in range(n_groups):
            g = ids_ref[pl.ds(j * 16, 16)]
            if j == 0:
                self_sh = plsc.load_gather(ids_ref, [shift_idx])
                p = jnp.where(lane0, carry, self_sh)
            else:
                p = ids_ref[pl.ds(j * 16 - 1, 16)]
            count_group(g, p, local_base + j * 16)
        return plsc.load_gather(
            ids_ref, [jnp.full((16,), n_groups * 16 - 1, jnp.int32)])

    issue(0, 0)

    def step(g_it, carry):
        for b in range(2):
            i = g_it * 2 + b
            wait_slot(b)
            pltpu.async_copy(rows_v[b], acc_sh.at[idx_v[b]], sem_s[b], add=True)
            carry = count_chunk(idx_v[b], i * CHUNK, N_GROUPS, carry)

            @pl.when(i >= 1)
            def _():
                wait_scatter(b ^ 1)

            @pl.when(i + 1 < N_CHUNKS)
            def _():
                issue(i + 1, b ^ 1)
        return carry

    carry = lax.fori_loop(0, N_CHUNKS // 2, step, carry0)
    wait_scatter((N_CHUNKS - 1) % 2)

    rem_base = row_base + N_CHUNKS * CHUNK
    pltpu.sync_copy(ids_hbm.at[pl.ds(rem_base, REM)], idx_r)
    pltpu.sync_copy(x_hbm.at[pl.ds(rem_base, REM)], rows_r)
    pltpu.sync_copy(rows_r, acc_sh.at[idx_r], add=True)
    lastid = count_chunk(idx_r, N_CHUNKS * CHUNK, REM // 16, carry)
    plsc.addupdate_scatter(cnt_v, [lastid],
                           jnp.full((16,), float(ROWS_PER_TILE), jnp.float32),
                           mask=lane0)

    pltpu.sync_copy(cnt_v, out_c.at[pl.ds(w * N_SEG_PAD, N_SEG_PAD)])

    plsc.subcore_barrier()

    pltpu.sync_copy(acc_sh.at[pl.ds(seg0, SEG_PER_TILE)],
                    out_s.at[pl.ds(c * N_SEG_PAD + seg0, SEG_PER_TILE)])


_sc_accumulate = functools.partial(
    pl.kernel,
    out_type=(
        jax.ShapeDtypeStruct((N_CORES * N_SEG_PAD, N_FEAT), jnp.float32),
        jax.ShapeDtypeStruct((N_TILES * N_SEG_PAD,), jnp.float32),
    ),
    mesh=plsc.VectorSubcoreMesh(core_axis_name="c", subcore_axis_name="s"),
    compiler_params=pltpu.CompilerParams(needs_layout_passes=False),
    scratch_types=(
        pltpu.VMEM((CHUNK,), jnp.int32),
        pltpu.VMEM((CHUNK,), jnp.int32),
        pltpu.VMEM((CHUNK, N_FEAT), jnp.float32),
        pltpu.VMEM((CHUNK, N_FEAT), jnp.float32),
        pltpu.VMEM((REM,), jnp.int32),
        pltpu.VMEM((REM, N_FEAT), jnp.float32),
        pltpu.VMEM((N_SEG_PAD,), jnp.float32),
        pltpu.SemaphoreType.DMA,
        pltpu.SemaphoreType.DMA,
        pltpu.SemaphoreType.DMA,
        pltpu.SemaphoreType.DMA,
        pltpu.SemaphoreType.DMA,
        pltpu.SemaphoreType.DMA,
        pltpu.VMEM_SHARED((N_SEG_PAD, N_FEAT), jnp.float32),
    ),
)(_sc_body)


def _combine_body(ps_ref, pc_ref, o_ref):
    sums = ps_ref[0] + ps_ref[1]
    cnt = jnp.sum(pc_ref[...], axis=0)
    o_ref[...] = sums / jnp.maximum(cnt, 1.0)[:, None]


_combine = pl.pallas_call(
    _combine_body,
    grid=(N_SEG_PAD // 1024,),
    in_specs=[
        pl.BlockSpec((N_CORES, 1024, N_FEAT), lambda j: (0, j, 0)),
        pl.BlockSpec((N_TILES, 1024), lambda j: (0, j)),
    ],
    out_specs=pl.BlockSpec((1024, N_FEAT), lambda j: (j, 0)),
    out_shape=jax.ShapeDtypeStruct((N_SEG_PAD, N_FEAT), jnp.float32),
)


@jax.jit
def kernel(inputs, unq_inv):
    ids = unq_inv.astype(jnp.int32)
    zs = jnp.zeros((SEG_PER_TILE, N_FEAT), jnp.float32)
    z1 = jnp.zeros((N_SEG_PAD,), jnp.float32)
    sums, cnts = _sc_accumulate(inputs, ids, zs, z1)
    padded = _combine(sums.reshape(N_CORES, N_SEG_PAD, N_FEAT),
                      cnts.reshape(N_TILES, N_SEG_PAD))
    return padded[:N_SEG]


# speedup vs baseline: 1.6197x; 1.0331x over previous
import functools

import jax
import jax.numpy as jnp
from jax import lax
from jax.experimental import pallas as pl
from jax.experimental.pallas import tpu as pltpu
from jax.experimental.pallas import tpu_sc as plsc

N_ROWS = 320000
N_FEAT = 128
N_SEG = 10000
N_CORES = 2
N_SUBCORES = 16
N_TILES = N_CORES * N_SUBCORES
ROWS_PER_TILE = N_ROWS // N_TILES
CHUNK = 128
N_CHUNKS = ROWS_PER_TILE // CHUNK
REM = ROWS_PER_TILE - N_CHUNKS * CHUNK
N_SEG_PAD = 10240
SEG_PER_TILE = N_SEG_PAD // N_SUBCORES
N_GROUPS = CHUNK // 16


def _sc_body(x_hbm, ids_hbm, out_s, out_c,
             idx_a, idx_b, rows_a, rows_b, idx_r, rows_r,
             cnt_v,
             sem_ia, sem_ib, sem_ra, sem_rb, sem_sa, sem_sb,
             acc_sh):
    c = lax.axis_index("c")
    s = lax.axis_index("s")
    w = c * N_SUBCORES + s
    seg0 = s * SEG_PER_TILE
    row_base = w * ROWS_PER_TILE
    idx_v = (idx_a, idx_b)
    rows_v = (rows_a, rows_b)
    sem_i = (sem_ia, sem_ib)
    sem_r = (sem_ra, sem_rb)
    sem_s = (sem_sa, sem_sb)

    iota = lax.iota(jnp.int32, 16)
    shift_idx = jnp.maximum(iota - 1, 0)
    lane0 = iota == 0

    zvec = jnp.zeros((16,), jnp.float32)

    def zrow(r, _):
        for k in range(8):
            rows_a[r, pl.ds(k * 16, 16)] = zvec
        return 0

    lax.fori_loop(0, CHUNK, zrow, 0)
    for k in range(SEG_PER_TILE // CHUNK):
        pltpu.sync_copy(rows_a, acc_sh.at[pl.ds(seg0 + k * CHUNK, CHUNK)])

    def zcnt(j, _):
        for k in range(8):
            cnt_v[pl.ds(j * CHUNK + k * 16, 16)] = zvec
        return 0

    lax.fori_loop(0, N_SEG_PAD // CHUNK, zcnt, 0)
    pltpu.sync_copy(ids_hbm.at[pl.ds(row_base, 16)], idx_r)
    carry0 = plsc.load_gather(idx_r, [jnp.zeros((16,), jnp.int32)])
    plsc.subcore_barrier()

    def issue(i, b):
        base = row_base + i * CHUNK
        pltpu.async_copy(ids_hbm.at[pl.ds(base, CHUNK)], idx_v[b], sem_i[b])
        pltpu.async_copy(x_hbm.at[pl.ds(base, CHUNK)], rows_v[b], sem_r[b])

    def wait_slot(b):
        pltpu.make_async_copy(ids_hbm.at[pl.ds(0, CHUNK)], idx_v[b],
                              sem_i[b]).wait()
        pltpu.make_async_copy(x_hbm.at[pl.ds(0, CHUNK)], rows_v[b],
                              sem_r[b]).wait()

    def wait_scatter(b):
        pltpu.make_async_copy(rows_v[b], acc_sh.at[idx_v[b]], sem_s[b]).wait()

    def count_group(g, p, base_val):
        m = g != p
        val = (base_val + iota).astype(jnp.float32)
        plsc.addupdate_scatter(cnt_v, [g], -val, mask=m)
        plsc.addupdate_scatter(cnt_v, [p], val, mask=m)

    def count_chunk(ids_ref, local_base, n_groups, carry):
        for j in range(n_groups):
            g = ids_ref[pl.ds(j * 16, 16)]
            if j == 0:
                self_sh = plsc.load_gather(ids_ref, [shift_idx])
                p = jnp.where(lane0, carry, self_sh)
            else:
                p = ids_ref[pl.ds(j * 16 - 1, 16)]
            count_group(g, p, local_base + j * 16)
        return plsc.load_gather(
            ids_ref, [jnp.full((16,), n_groups * 16 - 1, jnp.int32)])

    issue(0, 0)

    def step(g_it, carry):
        for b in range(2):
            i = g_it * 2 + b
            wait_slot(b)
            pltpu.async_copy(rows_v[b], acc_sh.at[idx_v[b]], sem_s[b], add=True)
            carry = count_chunk(idx_v[b], i * CHUNK, N_GROUPS, carry)

            @pl.when(i >= 1)
            def _():
                wait_scatter(b ^ 1)

            @pl.when(i + 1 < N_CHUNKS)
            def _():
                issue(i + 1, b ^ 1)
        return carry

    carry = lax.fori_loop(0, N_CHUNKS // 2, step, carry0)
    wait_scatter((N_CHUNKS - 1) % 2)

    rem_base = row_base + N_CHUNKS * CHUNK
    pltpu.sync_copy(ids_hbm.at[pl.ds(rem_base, REM)], idx_r)
    pltpu.sync_copy(x_hbm.at[pl.ds(rem_base, REM)], rows_r)
    pltpu.sync_copy(rows_r, acc_sh.at[idx_r], add=True)
    lastid = count_chunk(idx_r, N_CHUNKS * CHUNK, REM // 16, carry)
    plsc.addupdate_scatter(cnt_v, [lastid],
                           jnp.full((16,), float(ROWS_PER_TILE), jnp.float32),
                           mask=lane0)

    pltpu.sync_copy(cnt_v, out_c.at[pl.ds(w * N_SEG_PAD, N_SEG_PAD)])

    plsc.subcore_barrier()

    pltpu.sync_copy(acc_sh.at[pl.ds(seg0, SEG_PER_TILE)],
                    out_s.at[pl.ds(c * N_SEG_PAD + seg0, SEG_PER_TILE)])


_sc_accumulate = functools.partial(
    pl.kernel,
    out_type=(
        jax.ShapeDtypeStruct((N_CORES * N_SEG_PAD, N_FEAT), jnp.float32),
        jax.ShapeDtypeStruct((N_TILES * N_SEG_PAD,), jnp.float32),
    ),
    mesh=plsc.VectorSubcoreMesh(core_axis_name="c", subcore_axis_name="s"),
    compiler_params=pltpu.CompilerParams(needs_layout_passes=False),
    scratch_types=(
        pltpu.VMEM((CHUNK,), jnp.int32),
        pltpu.VMEM((CHUNK,), jnp.int32),
        pltpu.VMEM((CHUNK, N_FEAT), jnp.float32),
        pltpu.VMEM((CHUNK, N_FEAT), jnp.float32),
        pltpu.VMEM((REM,), jnp.int32),
        pltpu.VMEM((REM, N_FEAT), jnp.float32),
        pltpu.VMEM((N_SEG_PAD,), jnp.float32),
        pltpu.SemaphoreType.DMA,
        pltpu.SemaphoreType.DMA,
        pltpu.SemaphoreType.DMA,
        pltpu.SemaphoreType.DMA,
        pltpu.SemaphoreType.DMA,
        pltpu.SemaphoreType.DMA,
        pltpu.VMEM_SHARED((N_SEG_PAD, N_FEAT), jnp.float32),
    ),
)(_sc_body)


def _combine_body(ps_ref, pc_ref, o_ref):
    sums = ps_ref[0] + ps_ref[1]
    cnt = jnp.sum(pc_ref[...], axis=0)
    o_ref[...] = sums / jnp.maximum(cnt, 1.0)[:, None]


_combine = pl.pallas_call(
    _combine_body,
    grid=(N_SEG_PAD // 1024,),
    in_specs=[
        pl.BlockSpec((N_CORES, 1024, N_FEAT), lambda j: (0, j, 0)),
        pl.BlockSpec((N_TILES, 1024), lambda j: (0, j)),
    ],
    out_specs=pl.BlockSpec((1024, N_FEAT), lambda j: (j, 0)),
    out_shape=jax.ShapeDtypeStruct((N_SEG_PAD, N_FEAT), jnp.float32),
)


@jax.jit
def kernel(inputs, unq_inv):
    ids = unq_inv.astype(jnp.int32)
    sums, cnts = _sc_accumulate(inputs, ids)
    padded = _combine(sums.reshape(N_CORES, N_SEG_PAD, N_FEAT),
                      cnts.reshape(N_TILES, N_SEG_PAD))
    return padded[:N_SEG]
